# baseline (speedup 1.0000x reference)
import jax
import jax.numpy as jnp
from jax import lax
from jax.experimental import pallas as pl
from jax.experimental.pallas import tpu as pltpu

VOCAB = 100000
DIM = 300
MAX_NORM = 1.0
BATCH = 1024

_TV = 2048
_NVA = 48
_NBUF = 4


def _gather_body(idx_ref, emb_ref, out_ref, xs_ref, sem):
    def issue(j, _):
        pltpu.make_async_copy(
            emb_ref.at[pl.ds(idx_ref[j], 1)], xs_ref.at[pl.ds(j, 1)], sem
        ).start()
        return 0

    lax.fori_loop(0, BATCH, issue, 0)

    def drain(j, _):
        pltpu.make_async_copy(
            emb_ref.at[pl.ds(0, 1)], xs_ref.at[pl.ds(0, 1)], sem
        ).wait()
        return 0

    lax.fori_loop(0, BATCH, drain, 0)
    x = xs_ref[...]
    ss = jnp.sum(x * x, axis=1, keepdims=True)
    scale = jnp.minimum(1.0, MAX_NORM * lax.rsqrt(jnp.maximum(ss, 1e-14)))
    out_ref[...] = (x * scale).astype(jnp.bfloat16)


@jax.jit
def _gather_clip(idx, emb):
    grid_spec = pltpu.PrefetchScalarGridSpec(
        num_scalar_prefetch=1,
        grid=(1,),
        in_specs=[pl.BlockSpec(memory_space=pltpu.MemorySpace.HBM)],
        out_specs=pl.BlockSpec((BATCH, DIM), lambda i, idx_ref: (0, 0)),
        scratch_shapes=[
            pltpu.VMEM((BATCH, DIM), jnp.float32),
            pltpu.SemaphoreType.DMA,
        ],
    )
    return pl.pallas_call(
        _gather_body,
        grid_spec=grid_spec,
        out_shape=jax.ShapeDtypeStruct((BATCH, DIM), jnp.bfloat16),
    )(idx, emb)


def _mm_body(x_ref, w_ref, b_ref, o_ref, scr, *sems):
    i = pl.program_id(0)

    def cp(step, buf):
        return pltpu.make_async_copy(
            scr.at[buf], o_ref.at[:, pl.ds(step * _TV, _TV)], sems[buf]
        )

    acc = lax.dot_general(
        x_ref[...],
        w_ref[...].astype(jnp.bfloat16),
        (((1,), (1,)), ((), ())),
        preferred_element_type=jnp.float32,
    )
    val = acc + b_ref[...]

    for b in range(_NBUF):

        @pl.when(i % _NBUF == b)
        def _(b=b):
            @pl.when(i >= _NBUF)
            def _():
                cp(i - _NBUF, b).wait()

            scr[b] = val
            cp(i, b).start()

    @pl.when(i == _NVA - 1)
    def _():
        for d in range(1, _NBUF):
            cp(i - d, (_NVA - 1 - d) % _NBUF).wait()
        cp(i, (_NVA - 1) % _NBUF).wait()


@jax.jit
def _tc_project_main(x, W, b2d):
    return pl.pallas_call(
        _mm_body,
        grid=(_NVA,),
        in_specs=[
            pl.BlockSpec((BATCH, DIM), lambda i: (0, 0)),
            pl.BlockSpec((_TV, DIM), lambda i: (i, 0)),
            pl.BlockSpec((1, _TV), lambda i: (0, i)),
        ],
        out_specs=pl.BlockSpec(memory_space=pltpu.MemorySpace.HBM),
        out_shape=jax.ShapeDtypeStruct((BATCH, VOCAB), jnp.float32),
        scratch_shapes=[pltpu.VMEM((_NBUF, BATCH, _TV), jnp.float32)]
        + [pltpu.SemaphoreType.DMA] * _NBUF,
    )(x, W, b2d)


def _tail_body(x_ref, w_ref, b_ref, y_ref, o_ref):
    del y_ref
    acc = lax.dot_general(
        x_ref[...],
        w_ref[...].astype(jnp.bfloat16),
        (((1,), (1,)), ((), ())),
        preferred_element_type=jnp.float32,
    )
    o_ref[...] = acc + b_ref[...]


@jax.jit
def _tc_project_tail(x, W, b2d, y):
    return pl.pallas_call(
        _tail_body,
        grid=(1,),
        in_specs=[
            pl.BlockSpec((BATCH, DIM), lambda i: (0, 0)),
            pl.BlockSpec((_TV, DIM), lambda i: (_NVA, 0)),
            pl.BlockSpec((1, _TV), lambda i: (0, _NVA)),
            pl.BlockSpec(memory_space=pltpu.MemorySpace.HBM),
        ],
        out_specs=pl.BlockSpec((BATCH, _TV), lambda i: (0, _NVA)),
        out_shape=jax.ShapeDtypeStruct((BATCH, VOCAB), jnp.float32),
        input_output_aliases={3: 0},
    )(x, W, b2d, y)


def kernel(inputs_, emb, W, b):
    idx = inputs_.astype(jnp.int32)
    x = _gather_clip(idx, emb)
    b2d = b.reshape(1, VOCAB)
    y = _tc_project_main(x, W, b2d)
    return _tc_project_tail(x, W, b2d, y)


# speedup vs baseline: 1.0067x; 1.0067x over previous
import jax
import jax.numpy as jnp
from jax import lax
from jax.experimental import pallas as pl
from jax.experimental.pallas import tpu as pltpu

VOCAB = 100000
DIM = 300
MAX_NORM = 1.0
BATCH = 1024

_RPB = 64
_NB = BATCH // _RPB

_TV = 2048
_NV = (VOCAB + _TV - 1) // _TV


def _gather_body(idx_ref, emb_ref, out_ref, xs_ref, sem):
    def issue(j8, _):
        for u in range(8):
            j = j8 * 8 + u
            pltpu.make_async_copy(
                emb_ref.at[pl.ds(idx_ref[j], 1)], xs_ref.at[pl.ds(j, 1)], sem
            ).start()
        return 0

    lax.fori_loop(0, BATCH // 8, issue, 0)

    def drain(j, _):
        pltpu.make_async_copy(
            emb_ref.at[pl.ds(0, 1)], xs_ref.at[pl.ds(0, 1)], sem
        ).wait()
        return 0

    lax.fori_loop(0, BATCH, drain, 0)
    x = xs_ref[...]
    ss = jnp.sum(x * x, axis=1, keepdims=True)
    scale = jnp.minimum(1.0, MAX_NORM * lax.rsqrt(jnp.maximum(ss, 1e-14)))
    out_ref[...] = (x * scale).astype(jnp.bfloat16)


@jax.jit
def _gather_clip(idx, emb):
    grid_spec = pltpu.PrefetchScalarGridSpec(
        num_scalar_prefetch=1,
        grid=(1,),
        in_specs=[pl.BlockSpec(memory_space=pltpu.MemorySpace.HBM)],
        out_specs=pl.BlockSpec((BATCH, DIM), lambda i, idx_ref: (0, 0)),
        scratch_shapes=[
            pltpu.VMEM((BATCH, DIM), jnp.float32),
            pltpu.SemaphoreType.DMA,
        ],
    )
    return pl.pallas_call(
        _gather_body,
        grid_spec=grid_spec,
        out_shape=jax.ShapeDtypeStruct((BATCH, DIM), jnp.bfloat16),
    )(idx, emb)


def _mm_body(x_ref, w_ref, b_ref, o_ref):
    xb = x_ref[...]
    wb = w_ref[...].astype(jnp.bfloat16)
    acc = lax.dot_general(
        xb, wb, (((1,), (1,)), ((), ())), preferred_element_type=jnp.float32
    )
    o_ref[...] = acc + b_ref[...]


@jax.jit
def _tc_project(x, W, b2d):
    return pl.pallas_call(
        _mm_body,
        grid=(_NV,),
        in_specs=[
            pl.BlockSpec((BATCH, DIM), lambda i: (0, 0)),
            pl.BlockSpec((_TV, DIM), lambda i: (i, 0)),
            pl.BlockSpec((1, _TV), lambda i: (0, i)),
        ],
        out_specs=pl.BlockSpec((BATCH, _TV), lambda i: (0, i)),
        out_shape=jax.ShapeDtypeStruct((BATCH, VOCAB), jnp.float32),
    )(x, W, b2d)


def kernel(inputs_, emb, W, b):
    idx = inputs_.astype(jnp.int32)
    x = _gather_clip(idx, emb)
    return _tc_project(x, W, b.reshape(1, VOCAB))


# speedup vs baseline: 1.0134x; 1.0066x over previous
import jax
import jax.numpy as jnp
from jax import lax
from jax.experimental import pallas as pl
from jax.experimental.pallas import tpu as pltpu

VOCAB = 100000
DIM = 300
MAX_NORM = 1.0
BATCH = 1024

_RPB = 64
_NB = BATCH // _RPB

_TV = 4096
_NV = (VOCAB + _TV - 1) // _TV


def _gather_body(idx_ref, emb_ref, out_ref, xs_ref, sem):
    def issue(j8, _):
        for u in range(8):
            j = j8 * 8 + u
            pltpu.make_async_copy(
                emb_ref.at[pl.ds(idx_ref[j], 1)], xs_ref.at[pl.ds(j, 1)], sem
            ).start()
        return 0

    lax.fori_loop(0, BATCH // 8, issue, 0)

    def drain(j, _):
        pltpu.make_async_copy(
            emb_ref.at[pl.ds(0, 1)], xs_ref.at[pl.ds(0, 1)], sem
        ).wait()
        return 0

    lax.fori_loop(0, BATCH, drain, 0)
    x = xs_ref[...]
    ss = jnp.sum(x * x, axis=1, keepdims=True)
    scale = jnp.minimum(1.0, MAX_NORM * lax.rsqrt(jnp.maximum(ss, 1e-14)))
    out_ref[...] = (x * scale).astype(jnp.bfloat16)


@jax.jit
def _gather_clip(idx, emb):
    grid_spec = pltpu.PrefetchScalarGridSpec(
        num_scalar_prefetch=1,
        grid=(1,),
        in_specs=[pl.BlockSpec(memory_space=pltpu.MemorySpace.HBM)],
        out_specs=pl.BlockSpec((BATCH, DIM), lambda i, idx_ref: (0, 0)),
        scratch_shapes=[
            pltpu.VMEM((BATCH, DIM), jnp.float32),
            pltpu.SemaphoreType.DMA,
        ],
    )
    return pl.pallas_call(
        _gather_body,
        grid_spec=grid_spec,
        out_shape=jax.ShapeDtypeStruct((BATCH, DIM), jnp.bfloat16),
    )(idx, emb)


def _mm_body(x_ref, w_ref, b_ref, o_ref):
    xb = x_ref[...]
    wb = w_ref[...].astype(jnp.bfloat16)
    acc = lax.dot_general(
        xb, wb, (((1,), (1,)), ((), ())), preferred_element_type=jnp.float32
    )
    o_ref[...] = acc + b_ref[...]


@jax.jit
def _tc_project(x, W, b2d):
    return pl.pallas_call(
        _mm_body,
        grid=(_NV,),
        in_specs=[
            pl.BlockSpec((BATCH, DIM), lambda i: (0, 0)),
            pl.BlockSpec((_TV, DIM), lambda i: (i, 0)),
            pl.BlockSpec((1, _TV), lambda i: (0, i)),
        ],
        out_specs=pl.BlockSpec((BATCH, _TV), lambda i: (0, i)),
        out_shape=jax.ShapeDtypeStruct((BATCH, VOCAB), jnp.float32),
    )(x, W, b2d)


def kernel(inputs_, emb, W, b):
    idx = inputs_.astype(jnp.int32)
    x = _gather_clip(idx, emb)
    return _tc_project(x, W, b.reshape(1, VOCAB))
